# Initial kernel scaffold; baseline (speedup 1.0000x reference)
import jax
import jax.numpy as jnp
from jax.experimental import pallas as pl


def kernel(x, edge_index, batch, Wl1, Wr1, att1, b1, gw1, gb1, gm1, Wl2, Wr2, att2, b2, gw2, gb2, gm2, Wl3, Wr3, att3, b3, gw3, gb3, gm3, Wlin, blin):
    raise NotImplementedError("write your pallas kernel here")


# speedup vs baseline: 42.9089x; 42.9089x over previous
import functools

import jax
import jax.numpy as jnp
from jax import lax
from jax.experimental import pallas as pl
from jax.experimental.pallas import tpu as pltpu
from jax.experimental.pallas import tpu_sc as plsc

NC, NS, L = 2, 16, 16
NW = NC * NS
NG = 64
HC = 32
K = 80


def _make_edge_kernel(n, e, store_ex):
    epw = e // NW
    nchunks = epw // K
    npt = n // NS
    G = K // L
    n8 = -(-n // (8 * NS * 8)) * (NS * 8)
    npt8 = n8 // NS
    assert n % (NS * 8) == 0 and epw % K == 0
    mesh = plsc.VectorSubcoreMesh(core_axis_name="c", subcore_axis_name="s",
                                  num_cores=NC, num_subcores=NS)

    out_type = [
        jax.ShapeDtypeStruct((NC, 2, n, L), jnp.float32),
        jax.ShapeDtypeStruct((NC, n8, L), jnp.float32),
    ]
    if store_ex:
        out_type.append(jax.ShapeDtypeStruct((e,), jnp.float32))
        out_type.append(jax.ShapeDtypeStruct((e,), jnp.float32))

    scratch = [
        pltpu.VMEM((K,), jnp.int32),
        pltpu.VMEM((K,), jnp.int32),
        pltpu.VMEM((K, L), jnp.float32),
        pltpu.VMEM((K, L), jnp.float32),
        pltpu.VMEM((K, L), jnp.float32),
        pltpu.VMEM((K, L), jnp.float32),
        pltpu.VMEM((K,), jnp.int32),
        pltpu.VMEM((K,), jnp.float32),
        pltpu.VMEM((2, L), jnp.float32),
        pltpu.VMEM_SHARED((n, L), jnp.float32),
        pltpu.VMEM_SHARED((n8, L), jnp.float32),
        pltpu.SemaphoreType.DMA,
        pltpu.SemaphoreType.DMA,
    ]

    def body(*refs):
        if store_ex:
            (src_h, dst_h, xl0, xl1, xr0, xr1, att, zs, s_out, d_out,
             ex0_out, ex1_out,
             src_v, dst_v, xl_v, xr_v, s_v, d_v, dsh_v, ex_v, att_v,
             s_sh, d_sh, sem1, sem2) = refs
            ex_outs = (ex0_out, ex1_out)
        else:
            (src_h, dst_h, xl0, xl1, xr0, xr1, att, zs, s_out, d_out,
             src_v, dst_v, xl_v, xr_v, s_v, d_v, dsh_v, ex_v, att_v,
             s_sh, d_sh, sem1, sem2) = refs
            ex_outs = None
        cid = lax.axis_index("c")
        sid = lax.axis_index("s")
        wid = cid * NS + sid
        row0 = pl.multiple_of(sid * npt, 8)
        row8 = pl.multiple_of(sid * npt8, 8)

        pltpu.sync_copy(att, att_v)
        pltpu.sync_copy(zs.at[pl.ds(row0, npt), :], s_sh.at[pl.ds(row0, npt), :])
        pltpu.sync_copy(zs.at[pl.ds(0, npt8), :], d_sh.at[pl.ds(row8, npt8), :])

        rows0 = lax.iota(jnp.int32, L)
        zero16 = jnp.zeros((L,), jnp.float32)
        for j in range(K):
            d_v[j, :] = zero16
        plsc.subcore_barrier()

        for h in range(2):
            xlt = xl0 if h == 0 else xl1
            xrt = xr0 if h == 0 else xr1
            att_row = att_v[h, :]
            att_s = [att_row[c] for c in range(L)]

            def chunk(i, _, xlt=xlt, xrt=xrt, att_s=att_s, h=h):
                base = pl.multiple_of(wid * epw + i * K, 8)
                pltpu.sync_copy(src_h.at[pl.ds(base, K)], src_v)
                pltpu.sync_copy(dst_h.at[pl.ds(base, K)], dst_v)
                cp1 = pltpu.async_copy(xlt.at[src_v], xl_v, sem1)
                cp2 = pltpu.async_copy(xrt.at[dst_v], xr_v, sem2)
                cp1.wait()
                cp2.wait()
                cols_sav = []
                for g in range(G):
                    ridx = rows0 + (g * L)
                    acc = zero16
                    cols = []
                    for c in range(L):
                        cc = jnp.full((L,), c, jnp.int32)
                        a = plsc.load_gather(xl_v, [ridx, cc])
                        b = plsc.load_gather(xr_v, [ridx, cc])
                        z = a + b
                        zl = jnp.maximum(z, 0.2 * z)
                        acc = acc + zl * att_s[c]
                        cols.append(a)
                    ex = jnp.exp(acc)
                    ex_v[pl.ds(g * L, L)] = ex
                    for c in range(L):
                        cc = jnp.full((L,), c, jnp.int32)
                        plsc.store_scatter(s_v, [ridx, cc], ex * cols[c])
                    dvec = dst_v[pl.ds(g * L, L)]
                    colv = ((dvec & 7) << 1) + h
                    plsc.store_scatter(d_v, [ridx, colv], ex)
                    dsh_v[pl.ds(g * L, L)] = dvec >> 3
                    cols_sav.append((ridx, colv))
                pltpu.sync_copy(s_v, s_sh.at[dst_v], add=True)
                pltpu.sync_copy(d_v, d_sh.at[dsh_v], add=True)
                for ridx, colv in cols_sav:
                    plsc.store_scatter(d_v, [ridx, colv], zero16)
                if store_ex:
                    pltpu.sync_copy(ex_v, ex_outs[h].at[pl.ds(base, K)])
                return 0

            lax.fori_loop(0, nchunks, chunk, 0)
            plsc.subcore_barrier()
            pltpu.sync_copy(s_sh.at[pl.ds(row0, npt), :],
                            s_out.at[cid, h, pl.ds(row0, npt), :])
            plsc.subcore_barrier()
            if h == 0:
                pltpu.sync_copy(zs.at[pl.ds(row0, npt), :],
                                s_sh.at[pl.ds(row0, npt), :])
                plsc.subcore_barrier()
        pltpu.sync_copy(d_sh.at[pl.ds(row8, npt8), :],
                        d_out.at[cid, pl.ds(row8, npt8), :])

    return pl.kernel(body, out_type=out_type, mesh=mesh,
                     scratch_types=scratch,
                     compiler_params=pltpu.CompilerParams(
                         needs_layout_passes=False,
                         use_tc_tiling_on_sc=False))


def _make_alpha_kernel(n, e):
    epw = e // NW
    nchunks = epw // K
    G = K // L
    mesh = plsc.VectorSubcoreMesh(core_axis_name="c", subcore_axis_name="s",
                                  num_cores=NC, num_subcores=NS)

    scratch = [
        pltpu.VMEM((K,), jnp.int32),
        pltpu.VMEM((K, L), jnp.float32),
        pltpu.VMEM((K,), jnp.float32),
        pltpu.VMEM((K,), jnp.float32),
        pltpu.VMEM((K, 2), jnp.float32),
        pltpu.SemaphoreType.DMA,
    ]

    def body(dst_h, ex0_hbm, ex1_hbm, den_hbm, a3_out,
             dst_v, den_v, e0_v, e1_v, al_v, sem):
        cid = lax.axis_index("c")
        sid = lax.axis_index("s")
        wid = cid * NS + sid
        rows0 = lax.iota(jnp.int32, L)

        def chunk(i, _):
            base = pl.multiple_of(wid * epw + i * K, 8)
            pltpu.sync_copy(dst_h.at[pl.ds(base, K)], dst_v)
            cp = pltpu.async_copy(den_hbm.at[dst_v], den_v, sem)
            pltpu.sync_copy(ex0_hbm.at[pl.ds(base, K)], e0_v)
            pltpu.sync_copy(ex1_hbm.at[pl.ds(base, K)], e1_v)
            cp.wait()
            for g in range(G):
                ridx = rows0 + (g * L)
                for h in range(2):
                    hh = jnp.full((L,), h, jnp.int32)
                    den = plsc.load_gather(den_v, [ridx, hh])
                    ev = e0_v if h == 0 else e1_v
                    exg = ev[pl.ds(g * L, L)]
                    al = exg / (den + 1e-16)
                    plsc.store_scatter(al_v, [ridx, hh], al)
            pltpu.sync_copy(al_v, a3_out.at[pl.ds(base, K), :])
            return 0

        lax.fori_loop(0, nchunks, chunk, 0)

    return pl.kernel(
        body, out_type=jax.ShapeDtypeStruct((e, 2), jnp.float32),
        mesh=mesh, scratch_types=scratch,
        compiler_params=pltpu.CompilerParams(
            needs_layout_passes=False, use_tc_tiling_on_sc=False))


def _tables(xin, Wl, Wr, bn, n_pad):
    n, din = xin.shape
    nb = n // bn

    def body(x_ref, wl_ref, wr_ref, o1, o2, o3, o4):
        xl = jnp.dot(x_ref[...], wl_ref[...], preferred_element_type=jnp.float32,
                 precision=lax.Precision.HIGHEST)
        xr = jnp.dot(x_ref[...], wr_ref[...], preferred_element_type=jnp.float32,
                 precision=lax.Precision.HIGHEST)
        o1[...] = xl[:, :L]
        o2[...] = xl[:, L:]
        o3[...] = xr[:, :L]
        o4[...] = xr[:, L:]

    return pl.pallas_call(
        body,
        grid=(nb,),
        in_specs=[
            pl.BlockSpec((bn, din), lambda j: (j, 0)),
            pl.BlockSpec((din, HC), lambda j: (0, 0)),
            pl.BlockSpec((din, HC), lambda j: (0, 0)),
        ],
        out_specs=[pl.BlockSpec((bn, L), lambda j: (j, 0))] * 4,
        out_shape=[jax.ShapeDtypeStruct((n_pad, L), jnp.float32)] * 4,
    )(xin, Wl, Wr)


def _onehot(bt):
    return (bt[:, None] == lax.broadcasted_iota(jnp.int32, (1, NG), 1)
            ).astype(jnp.float32)


def _combine(s_parts, d_parts, bias2, batch3, n, bn, relu, want_dtot):
    n_pad = s_parts.shape[2]
    nb = n // bn
    outs = [
        jax.ShapeDtypeStruct((n, HC), jnp.float32),
        jax.ShapeDtypeStruct((NG, HC), jnp.float32),
        jax.ShapeDtypeStruct((NG, HC), jnp.float32),
    ]
    if want_dtot:
        outs.append(jax.ShapeDtypeStruct((n_pad, L), jnp.float32))

    def body(s_ref, d_ref, b_ref, bt_ref, t_out, hsum_out, cnt_out,
             *maybe_dtot):
        j = pl.program_id(0)
        d = d_ref[0] + d_ref[1]
        parts = []
        for h in range(2):
            sh = s_ref[0, h] + s_ref[1, h]
            parts.append(sh / (d[:, h:h + 1] + 1e-16))
        t = jnp.concatenate(parts, axis=1) + b_ref[...]
        if relu:
            t = jnp.maximum(t, 0.0)
        t_out[...] = t
        oh = _onehot(bt_ref[0, 0])

        @pl.when(j == 0)
        def _():
            hsum_out[...] = jnp.zeros_like(hsum_out)
            cnt_out[...] = jnp.zeros_like(cnt_out)

        hsum_out[...] += jnp.dot(oh.T, t, preferred_element_type=jnp.float32,
                 precision=lax.Precision.HIGHEST)
        cnt_out[...] += jnp.dot(
            oh.T, jnp.ones((bn, HC), jnp.float32),
            preferred_element_type=jnp.float32,
                 precision=lax.Precision.HIGHEST)
        if want_dtot:
            maybe_dtot[0][...] = jnp.concatenate(
                [d, jnp.zeros((d.shape[0], L - 2), jnp.float32)], axis=1)

    out_specs = [
        pl.BlockSpec((bn, HC), lambda j: (j, 0)),
        pl.BlockSpec((NG, HC), lambda j: (0, 0)),
        pl.BlockSpec((NG, HC), lambda j: (0, 0)),
    ]
    if want_dtot:
        out_specs.append(pl.BlockSpec((bn, L), lambda j: (j, 0)))

    return pl.pallas_call(
        body,
        grid=(nb,),
        in_specs=[
            pl.BlockSpec((NC, 2, bn, L), lambda j: (0, 0, j, 0)),
            pl.BlockSpec((NC, bn, 2), lambda j: (0, j, 0)),
            pl.BlockSpec((1, HC), lambda j: (0, 0)),
            pl.BlockSpec((1, 1, bn), lambda j: (j, 0, 0)),
        ],
        out_specs=out_specs,
        out_shape=outs,
    )(s_parts, d_parts, bias2, batch3)


def _center(t, hsum, cnt, batch3, gm2, bn):
    n = t.shape[0]
    nb = n // bn

    def body(t_ref, hsum_ref, cnt_ref, bt_ref, gm_ref, y_out, vsum_out):
        j = pl.program_id(0)
        cnt = jnp.maximum(cnt_ref[...], 1.0)
        mean = hsum_ref[...] / cnt
        oh = _onehot(bt_ref[0, 0])
        mean_n = jnp.dot(oh, mean, preferred_element_type=jnp.float32,
                 precision=lax.Precision.HIGHEST)
        y = t_ref[...] - mean_n * gm_ref[...]
        y_out[...] = y

        @pl.when(j == 0)
        def _():
            vsum_out[...] = jnp.zeros_like(vsum_out)

        vsum_out[...] += jnp.dot(oh.T, y * y,
                                 preferred_element_type=jnp.float32,
                 precision=lax.Precision.HIGHEST)

    return pl.pallas_call(
        body,
        grid=(nb,),
        in_specs=[
            pl.BlockSpec((bn, HC), lambda j: (j, 0)),
            pl.BlockSpec((NG, HC), lambda j: (0, 0)),
            pl.BlockSpec((NG, HC), lambda j: (0, 0)),
            pl.BlockSpec((1, 1, bn), lambda j: (j, 0, 0)),
            pl.BlockSpec((1, HC), lambda j: (0, 0)),
        ],
        out_specs=[
            pl.BlockSpec((bn, HC), lambda j: (j, 0)),
            pl.BlockSpec((NG, HC), lambda j: (0, 0)),
        ],
        out_shape=[
            jax.ShapeDtypeStruct((n, HC), jnp.float32),
            jax.ShapeDtypeStruct((NG, HC), jnp.float32),
        ],
    )(t, hsum, cnt, batch3, gm2)


def _finish(y, vsum, cnt, batch3, gw2, gb2, Wl_next, Wr_next, bn, n_pad):
    n = y.shape[0]
    nb = n // bn
    last = Wl_next is None

    def body(y_ref, vsum_ref, cnt_ref, bt_ref, gw_ref, gb_ref, *rest):
        j = pl.program_id(0)
        cnt = jnp.maximum(cnt_ref[...], 1.0)
        var = vsum_ref[...] / cnt
        r = lax.rsqrt(var + 1e-5)
        oh = _onehot(bt_ref[0, 0])
        rn = jnp.dot(oh, r, preferred_element_type=jnp.float32,
                 precision=lax.Precision.HIGHEST)
        res = y_ref[...] * rn * gw_ref[...] + gb_ref[...]
        if last:
            (psum_out,) = rest

            @pl.when(j == 0)
            def _():
                psum_out[...] = jnp.zeros_like(psum_out)

            psum_out[...] += jnp.dot(oh.T, res,
                                     preferred_element_type=jnp.float32,
                 precision=lax.Precision.HIGHEST)
        else:
            wl_ref, wr_ref, o1, o2, o3, o4 = rest
            xl = jnp.dot(res, wl_ref[...], preferred_element_type=jnp.float32,
                 precision=lax.Precision.HIGHEST)
            xr = jnp.dot(res, wr_ref[...], preferred_element_type=jnp.float32,
                 precision=lax.Precision.HIGHEST)
            o1[...] = xl[:, :L]
            o2[...] = xl[:, L:]
            o3[...] = xr[:, :L]
            o4[...] = xr[:, L:]

    in_specs = [
        pl.BlockSpec((bn, HC), lambda j: (j, 0)),
        pl.BlockSpec((NG, HC), lambda j: (0, 0)),
        pl.BlockSpec((NG, HC), lambda j: (0, 0)),
        pl.BlockSpec((1, 1, bn), lambda j: (j, 0, 0)),
        pl.BlockSpec((1, HC), lambda j: (0, 0)),
        pl.BlockSpec((1, HC), lambda j: (0, 0)),
    ]
    args = [y, vsum, cnt, batch3, gw2, gb2]
    if last:
        out_specs = [pl.BlockSpec((NG, HC), lambda j: (0, 0))]
        out_shape = [jax.ShapeDtypeStruct((NG, HC), jnp.float32)]
    else:
        in_specs += [pl.BlockSpec((HC, HC), lambda j: (0, 0))] * 2
        args += [Wl_next, Wr_next]
        out_specs = [pl.BlockSpec((bn, L), lambda j: (j, 0))] * 4
        out_shape = [jax.ShapeDtypeStruct((n_pad, L), jnp.float32)] * 4

    return pl.pallas_call(body, grid=(nb,), in_specs=in_specs,
                          out_specs=out_specs, out_shape=out_shape)(*args)


def _head(psum, cnt, Wlin, blin):
    def body(p_ref, c_ref, w_ref, b_ref, pooled_out, o_out):
        pooled = p_ref[...] / jnp.maximum(c_ref[...], 1.0)
        pooled_out[...] = pooled
        o_out[...] = jnp.dot(pooled, w_ref[...],
                             preferred_element_type=jnp.float32,
                 precision=lax.Precision.HIGHEST) + b_ref[...]

    return pl.pallas_call(
        body,
        in_specs=[
            pl.BlockSpec((NG, HC), lambda: (0, 0)),
            pl.BlockSpec((NG, HC), lambda: (0, 0)),
            pl.BlockSpec((HC, 2), lambda: (0, 0)),
            pl.BlockSpec((1, 2), lambda: (0, 0)),
        ],
        out_specs=[
            pl.BlockSpec((NG, HC), lambda: (0, 0)),
            pl.BlockSpec((NG, 2), lambda: (0, 0)),
        ],
        out_shape=[
            jax.ShapeDtypeStruct((NG, HC), jnp.float32),
            jax.ShapeDtypeStruct((NG, 2), jnp.float32),
        ],
    )(psum, cnt, Wlin, blin)


def kernel(x, edge_index, batch, Wl1, Wr1, att1, b1, gw1, gb1, gm1,
           Wl2, Wr2, att2, b2, gw2, gb2, gm2,
           Wl3, Wr3, att3, b3, gw3, gb3, gm3, Wlin, blin):
    n = x.shape[0]
    e = edge_index.shape[1]
    bn = 2000
    nb = n // bn
    n_pad = -(-n // (NS * 8)) * (NS * 8)
    batch3 = batch.reshape(nb, 1, bn)
    src = edge_index[0]
    dst = edge_index[1]
    zs = jnp.zeros((n_pad, L), jnp.float32)

    edge1 = _make_edge_kernel(n_pad, e, store_ex=False)
    edge3 = _make_edge_kernel(n_pad, e, store_ex=True)
    alpha3 = _make_alpha_kernel(n_pad, e)

    tabs = _tables(x, Wl1, Wr1, bn, n_pad)
    layer_params = [
        (att1, b1, gw1, gb1, gm1, Wl2, Wr2),
        (att2, b2, gw2, gb2, gm2, Wl3, Wr3),
        (att3, b3, gw3, gb3, gm3, None, None),
    ]
    ex = dtot = psum = cnt = None
    for li, (att, bb, gw, gb, gm, Wln, Wrn) in enumerate(layer_params):
        last = li == 2
        ek = edge3 if last else edge1
        res = ek(src, dst, tabs[0], tabs[1], tabs[2], tabs[3],
                 att.reshape(2, L), zs)
        if last:
            s_parts, d_parts, ex0, ex1 = res
        else:
            s_parts, d_parts = res
        d_parts = d_parts.reshape(NC, -1, 2)
        cres = _combine(s_parts, d_parts, bb.reshape(1, HC), batch3, n, bn,
                        relu=not last, want_dtot=last)
        if last:
            t, hsum, cnt, dtot = cres
        else:
            t, hsum, cnt = cres
        y, vsum = _center(t, hsum, cnt, batch3, gm.reshape(1, HC), bn)
        fres = _finish(y, vsum, cnt, batch3, gw.reshape(1, HC),
                       gb.reshape(1, HC), Wln, Wrn, bn, n_pad)
        if last:
            (psum,) = fres
        else:
            tabs = fres

    a3 = alpha3(dst, ex0, ex1, dtot)
    pooled, o = _head(psum, cnt, Wlin, blin.reshape(1, 2))
    return (o, pooled, a3)


# speedup vs baseline: 50.5380x; 1.1778x over previous
import functools

import jax
import jax.numpy as jnp
from jax import lax
from jax.experimental import pallas as pl
from jax.experimental.pallas import tpu as pltpu
from jax.experimental.pallas import tpu_sc as plsc

NC, NS, L = 2, 16, 16
NW = NC * NS
NG = 64
HC = 32
K = 80


def _make_edge_kernel(n, e, store_ex):
    epw = e // NW
    nchunks = epw // K
    npt = n // NS
    G = K // L
    n8 = -(-n // (8 * NS * 8)) * (NS * 8)
    npt8 = n8 // NS
    npairs = (nchunks + 1) // 2
    assert n % (NS * 8) == 0 and epw % K == 0
    mesh = plsc.VectorSubcoreMesh(core_axis_name="c", subcore_axis_name="s",
                                  num_cores=NC, num_subcores=NS)

    out_type = [
        jax.ShapeDtypeStruct((NC, 2, n, L), jnp.float32),
        jax.ShapeDtypeStruct((NC, n8, L), jnp.float32),
    ]
    if store_ex:
        out_type.append(jax.ShapeDtypeStruct((e,), jnp.float32))
        out_type.append(jax.ShapeDtypeStruct((e,), jnp.float32))

    scratch = [
        pltpu.VMEM((2, K), jnp.int32),
        pltpu.VMEM((2, K), jnp.int32),
        pltpu.VMEM((K,), jnp.int32),
        pltpu.VMEM((K,), jnp.int32),
        pltpu.VMEM((K,), jnp.int32),
        pltpu.VMEM((K,), jnp.int32),
        pltpu.VMEM((2, K, L), jnp.float32),
        pltpu.VMEM((2, K, L), jnp.float32),
        pltpu.VMEM((2, K, L), jnp.float32),
        pltpu.VMEM((2, K, L), jnp.float32),
        pltpu.VMEM((2, K), jnp.float32),
        pltpu.VMEM((2, L), jnp.float32),
        pltpu.VMEM_SHARED((n, L), jnp.float32),
        pltpu.VMEM_SHARED((n8, L), jnp.float32),
        pltpu.SemaphoreType.DMA,
        pltpu.SemaphoreType.DMA,
    ]

    def body(*refs):
        if store_ex:
            (eic, xl0, xl1, xr0, xr1, att, zs, s_out, d_out,
             ex0_out, ex1_out,
             eiA, eiB, dwA, dwB, dhA, dhB, xl_v, xr_v, s_v, d_v, ex_v, att_v,
             s_sh, d_sh, sg0, sg1) = refs
            ex_outs = (ex0_out, ex1_out)
        else:
            (eic, xl0, xl1, xr0, xr1, att, zs, s_out, d_out,
             eiA, eiB, dwA, dwB, dhA, dhB, xl_v, xr_v, s_v, d_v, ex_v, att_v,
             s_sh, d_sh, sg0, sg1) = refs
            ex_outs = None
        sem_g = (sg0, sg1)
        ei = (eiA, eiB)
        dw = (dwA, dwB)
        dh = (dhA, dhB)
        cid = lax.axis_index("c")
        sid = lax.axis_index("s")
        wid = cid * NS + sid
        row0 = pl.multiple_of(sid * npt, 8)
        row8 = pl.multiple_of(sid * npt8, 8)

        pltpu.sync_copy(att, att_v)
        pltpu.sync_copy(zs.at[pl.ds(row0, npt), :], s_sh.at[pl.ds(row0, npt), :])
        pltpu.sync_copy(zs.at[pl.ds(0, npt8), :], d_sh.at[pl.ds(row8, npt8), :])

        rows0 = lax.iota(jnp.int32, L)
        zero16 = jnp.zeros((L,), jnp.float32)
        plsc.subcore_barrier()

        for h in range(2):
            xlt = xl0 if h == 0 else xl1
            xrt = xr0 if h == 0 else xr1
            att_row = att_v[h, :]
            att_s = [att_row[c] for c in range(L)]

            def fetch_and_gather(c, b, xlt=xlt, xrt=xrt):
                cix = wid * nchunks + c
                pltpu.sync_copy(eic.at[cix], ei[b])
                for g in range(G):
                    dw[b][pl.ds(g * L, L)] = ei[b][1, pl.ds(g * L, L)]
                pltpu.async_copy(xlt.at[ei[b].at[0]], xl_v.at[b], sem_g[b])
                pltpu.async_copy(xrt.at[dw[b]], xr_v.at[b], sem_g[b])

            def drain_g(b, xlt=xlt, xrt=xrt):
                pltpu.make_async_copy(xlt.at[ei[b].at[0]], xl_v.at[b],
                                      sem_g[b]).wait()
                pltpu.make_async_copy(xrt.at[dw[b]], xr_v.at[b],
                                      sem_g[b]).wait()

            def issue_scatters(c, b, h=h):
                pltpu.sync_copy(s_v.at[b], s_sh.at[dw[b]], add=True)
                pltpu.sync_copy(d_v.at[b], d_sh.at[dh[b]], add=True)
                if store_ex:
                    base = pl.multiple_of(wid * epw + c * K, 8)
                    pltpu.sync_copy(ex_v.at[b], ex_outs[h].at[pl.ds(base, K)])

            def compute(c, b, h=h, att_s=att_s):
                for jj in range(K):
                    d_v[b, jj, :] = zero16
                for g in range(G):
                    ridx = rows0 + (g * L)
                    acc = zero16
                    cols = []
                    for c16 in range(L):
                        cc = jnp.full((L,), c16, jnp.int32)
                        a = plsc.load_gather(xl_v.at[b], [ridx, cc])
                        bb = plsc.load_gather(xr_v.at[b], [ridx, cc])
                        z = a + bb
                        zl = jnp.maximum(z, 0.2 * z)
                        acc = acc + zl * att_s[c16]
                        cols.append(a)
                    ex = jnp.exp(acc)
                    ex_v[b, pl.ds(g * L, L)] = ex
                    for c16 in range(L):
                        cc = jnp.full((L,), c16, jnp.int32)
                        plsc.store_scatter(s_v.at[b], [ridx, cc], ex * cols[c16])
                    dvec = dw[b][pl.ds(g * L, L)]
                    colv = ((dvec & 7) << 1) + h
                    plsc.store_scatter(d_v.at[b], [ridx, colv], ex)
                    dh[b][pl.ds(g * L, L)] = dvec >> 3

            def half(c, b):
                drain_g(b)
                compute(c, b)

                @pl.when(c + 1 < nchunks)
                def _():
                    fetch_and_gather(c + 1, 1 - b)

                issue_scatters(c, b)

            def pair(j, _):
                c0 = 2 * j
                half(c0, 0)

                @pl.when(c0 + 1 < nchunks)
                def _():
                    half(c0 + 1, 1)

                return 0

            fetch_and_gather(0, 0)
            lax.fori_loop(0, npairs, pair, 0)
            plsc.subcore_barrier()
            pltpu.sync_copy(s_sh.at[pl.ds(row0, npt), :],
                            s_out.at[cid, h, pl.ds(row0, npt), :])
            plsc.subcore_barrier()
            if h == 0:
                pltpu.sync_copy(zs.at[pl.ds(row0, npt), :],
                                s_sh.at[pl.ds(row0, npt), :])
                plsc.subcore_barrier()
        pltpu.sync_copy(d_sh.at[pl.ds(row8, npt8), :],
                        d_out.at[cid, pl.ds(row8, npt8), :])

    return pl.kernel(body, out_type=out_type, mesh=mesh,
                     scratch_types=scratch,
                     compiler_params=pltpu.CompilerParams(
                         needs_layout_passes=False,
                         use_tc_tiling_on_sc=False))


def _make_alpha_kernel(n, e):
    epw = e // NW
    nchunks = epw // K
    G = K // L
    mesh = plsc.VectorSubcoreMesh(core_axis_name="c", subcore_axis_name="s",
                                  num_cores=NC, num_subcores=NS)

    scratch = [
        pltpu.VMEM((K,), jnp.int32),
        pltpu.VMEM((K, L), jnp.float32),
        pltpu.VMEM((K,), jnp.float32),
        pltpu.VMEM((K,), jnp.float32),
        pltpu.VMEM((K, 2), jnp.float32),
        pltpu.SemaphoreType.DMA,
    ]

    def body(dst_h, ex0_hbm, ex1_hbm, den_hbm, a3_out,
             dst_v, den_v, e0_v, e1_v, al_v, sem):
        cid = lax.axis_index("c")
        sid = lax.axis_index("s")
        wid = cid * NS + sid
        rows0 = lax.iota(jnp.int32, L)

        def chunk(i, _):
            base = pl.multiple_of(wid * epw + i * K, 8)
            pltpu.sync_copy(dst_h.at[pl.ds(base, K)], dst_v)
            cp = pltpu.async_copy(den_hbm.at[dst_v], den_v, sem)
            pltpu.sync_copy(ex0_hbm.at[pl.ds(base, K)], e0_v)
            pltpu.sync_copy(ex1_hbm.at[pl.ds(base, K)], e1_v)
            cp.wait()
            for g in range(G):
                ridx = rows0 + (g * L)
                for h in range(2):
                    hh = jnp.full((L,), h, jnp.int32)
                    den = plsc.load_gather(den_v, [ridx, hh])
                    ev = e0_v if h == 0 else e1_v
                    exg = ev[pl.ds(g * L, L)]
                    al = exg / (den + 1e-16)
                    plsc.store_scatter(al_v, [ridx, hh], al)
            pltpu.sync_copy(al_v, a3_out.at[pl.ds(base, K), :])
            return 0

        lax.fori_loop(0, nchunks, chunk, 0)

    return pl.kernel(
        body, out_type=jax.ShapeDtypeStruct((e, 2), jnp.float32),
        mesh=mesh, scratch_types=scratch,
        compiler_params=pltpu.CompilerParams(
            needs_layout_passes=False, use_tc_tiling_on_sc=False))


def _tables(xin, Wl, Wr, bn, n_pad):
    n, din = xin.shape
    nb = n // bn

    def body(x_ref, wl_ref, wr_ref, o1, o2, o3, o4):
        xl = jnp.dot(x_ref[...], wl_ref[...], preferred_element_type=jnp.float32,
                 precision=lax.Precision.HIGHEST)
        xr = jnp.dot(x_ref[...], wr_ref[...], preferred_element_type=jnp.float32,
                 precision=lax.Precision.HIGHEST)
        o1[...] = xl[:, :L]
        o2[...] = xl[:, L:]
        o3[...] = xr[:, :L]
        o4[...] = xr[:, L:]

    return pl.pallas_call(
        body,
        grid=(nb,),
        in_specs=[
            pl.BlockSpec((bn, din), lambda j: (j, 0)),
            pl.BlockSpec((din, HC), lambda j: (0, 0)),
            pl.BlockSpec((din, HC), lambda j: (0, 0)),
        ],
        out_specs=[pl.BlockSpec((bn, L), lambda j: (j, 0))] * 4,
        out_shape=[jax.ShapeDtypeStruct((n_pad, L), jnp.float32)] * 4,
    )(xin, Wl, Wr)


def _onehot(bt):
    return (bt[:, None] == lax.broadcasted_iota(jnp.int32, (1, NG), 1)
            ).astype(jnp.float32)


def _combine(s_parts, d_parts, bias2, batch3, n, bn, relu, want_dtot):
    n_pad = s_parts.shape[2]
    nb = n // bn
    outs = [
        jax.ShapeDtypeStruct((n, HC), jnp.float32),
        jax.ShapeDtypeStruct((NG, HC), jnp.float32),
        jax.ShapeDtypeStruct((NG, HC), jnp.float32),
    ]
    if want_dtot:
        outs.append(jax.ShapeDtypeStruct((n_pad, L), jnp.float32))

    def body(s_ref, d_ref, b_ref, bt_ref, t_out, hsum_out, cnt_out,
             *maybe_dtot):
        j = pl.program_id(0)
        d = d_ref[0] + d_ref[1]
        parts = []
        for h in range(2):
            sh = s_ref[0, h] + s_ref[1, h]
            parts.append(sh / (d[:, h:h + 1] + 1e-16))
        t = jnp.concatenate(parts, axis=1) + b_ref[...]
        if relu:
            t = jnp.maximum(t, 0.0)
        t_out[...] = t
        oh = _onehot(bt_ref[0, 0])

        @pl.when(j == 0)
        def _():
            hsum_out[...] = jnp.zeros_like(hsum_out)
            cnt_out[...] = jnp.zeros_like(cnt_out)

        hsum_out[...] += jnp.dot(oh.T, t, preferred_element_type=jnp.float32,
                 precision=lax.Precision.HIGHEST)
        cnt_out[...] += jnp.dot(
            oh.T, jnp.ones((bn, HC), jnp.float32),
            preferred_element_type=jnp.float32,
                 precision=lax.Precision.HIGHEST)
        if want_dtot:
            maybe_dtot[0][...] = jnp.concatenate(
                [d, jnp.zeros((d.shape[0], L - 2), jnp.float32)], axis=1)

    out_specs = [
        pl.BlockSpec((bn, HC), lambda j: (j, 0)),
        pl.BlockSpec((NG, HC), lambda j: (0, 0)),
        pl.BlockSpec((NG, HC), lambda j: (0, 0)),
    ]
    if want_dtot:
        out_specs.append(pl.BlockSpec((bn, L), lambda j: (j, 0)))

    return pl.pallas_call(
        body,
        grid=(nb,),
        in_specs=[
            pl.BlockSpec((NC, 2, bn, L), lambda j: (0, 0, j, 0)),
            pl.BlockSpec((NC, bn, 2), lambda j: (0, j, 0)),
            pl.BlockSpec((1, HC), lambda j: (0, 0)),
            pl.BlockSpec((1, 1, bn), lambda j: (j, 0, 0)),
        ],
        out_specs=out_specs,
        out_shape=outs,
    )(s_parts, d_parts, bias2, batch3)


def _center(t, hsum, cnt, batch3, gm2, bn):
    n = t.shape[0]
    nb = n // bn

    def body(t_ref, hsum_ref, cnt_ref, bt_ref, gm_ref, y_out, vsum_out):
        j = pl.program_id(0)
        cnt = jnp.maximum(cnt_ref[...], 1.0)
        mean = hsum_ref[...] / cnt
        oh = _onehot(bt_ref[0, 0])
        mean_n = jnp.dot(oh, mean, preferred_element_type=jnp.float32,
                 precision=lax.Precision.HIGHEST)
        y = t_ref[...] - mean_n * gm_ref[...]
        y_out[...] = y

        @pl.when(j == 0)
        def _():
            vsum_out[...] = jnp.zeros_like(vsum_out)

        vsum_out[...] += jnp.dot(oh.T, y * y,
                                 preferred_element_type=jnp.float32,
                 precision=lax.Precision.HIGHEST)

    return pl.pallas_call(
        body,
        grid=(nb,),
        in_specs=[
            pl.BlockSpec((bn, HC), lambda j: (j, 0)),
            pl.BlockSpec((NG, HC), lambda j: (0, 0)),
            pl.BlockSpec((NG, HC), lambda j: (0, 0)),
            pl.BlockSpec((1, 1, bn), lambda j: (j, 0, 0)),
            pl.BlockSpec((1, HC), lambda j: (0, 0)),
        ],
        out_specs=[
            pl.BlockSpec((bn, HC), lambda j: (j, 0)),
            pl.BlockSpec((NG, HC), lambda j: (0, 0)),
        ],
        out_shape=[
            jax.ShapeDtypeStruct((n, HC), jnp.float32),
            jax.ShapeDtypeStruct((NG, HC), jnp.float32),
        ],
    )(t, hsum, cnt, batch3, gm2)


def _finish(y, vsum, cnt, batch3, gw2, gb2, Wl_next, Wr_next, bn, n_pad):
    n = y.shape[0]
    nb = n // bn
    last = Wl_next is None

    def body(y_ref, vsum_ref, cnt_ref, bt_ref, gw_ref, gb_ref, *rest):
        j = pl.program_id(0)
        cnt = jnp.maximum(cnt_ref[...], 1.0)
        var = vsum_ref[...] / cnt
        r = lax.rsqrt(var + 1e-5)
        oh = _onehot(bt_ref[0, 0])
        rn = jnp.dot(oh, r, preferred_element_type=jnp.float32,
                 precision=lax.Precision.HIGHEST)
        res = y_ref[...] * rn * gw_ref[...] + gb_ref[...]
        if last:
            (psum_out,) = rest

            @pl.when(j == 0)
            def _():
                psum_out[...] = jnp.zeros_like(psum_out)

            psum_out[...] += jnp.dot(oh.T, res,
                                     preferred_element_type=jnp.float32,
                 precision=lax.Precision.HIGHEST)
        else:
            wl_ref, wr_ref, o1, o2, o3, o4 = rest
            xl = jnp.dot(res, wl_ref[...], preferred_element_type=jnp.float32,
                 precision=lax.Precision.HIGHEST)
            xr = jnp.dot(res, wr_ref[...], preferred_element_type=jnp.float32,
                 precision=lax.Precision.HIGHEST)
            o1[...] = xl[:, :L]
            o2[...] = xl[:, L:]
            o3[...] = xr[:, :L]
            o4[...] = xr[:, L:]

    in_specs = [
        pl.BlockSpec((bn, HC), lambda j: (j, 0)),
        pl.BlockSpec((NG, HC), lambda j: (0, 0)),
        pl.BlockSpec((NG, HC), lambda j: (0, 0)),
        pl.BlockSpec((1, 1, bn), lambda j: (j, 0, 0)),
        pl.BlockSpec((1, HC), lambda j: (0, 0)),
        pl.BlockSpec((1, HC), lambda j: (0, 0)),
    ]
    args = [y, vsum, cnt, batch3, gw2, gb2]
    if last:
        out_specs = [pl.BlockSpec((NG, HC), lambda j: (0, 0))]
        out_shape = [jax.ShapeDtypeStruct((NG, HC), jnp.float32)]
    else:
        in_specs += [pl.BlockSpec((HC, HC), lambda j: (0, 0))] * 2
        args += [Wl_next, Wr_next]
        out_specs = [pl.BlockSpec((bn, L), lambda j: (j, 0))] * 4
        out_shape = [jax.ShapeDtypeStruct((n_pad, L), jnp.float32)] * 4

    return pl.pallas_call(body, grid=(nb,), in_specs=in_specs,
                          out_specs=out_specs, out_shape=out_shape)(*args)


def _head(psum, cnt, Wlin, blin):
    def body(p_ref, c_ref, w_ref, b_ref, pooled_out, o_out):
        pooled = p_ref[...] / jnp.maximum(c_ref[...], 1.0)
        pooled_out[...] = pooled
        o_out[...] = jnp.dot(pooled, w_ref[...],
                             preferred_element_type=jnp.float32,
                 precision=lax.Precision.HIGHEST) + b_ref[...]

    return pl.pallas_call(
        body,
        in_specs=[
            pl.BlockSpec((NG, HC), lambda: (0, 0)),
            pl.BlockSpec((NG, HC), lambda: (0, 0)),
            pl.BlockSpec((HC, 2), lambda: (0, 0)),
            pl.BlockSpec((1, 2), lambda: (0, 0)),
        ],
        out_specs=[
            pl.BlockSpec((NG, HC), lambda: (0, 0)),
            pl.BlockSpec((NG, 2), lambda: (0, 0)),
        ],
        out_shape=[
            jax.ShapeDtypeStruct((NG, HC), jnp.float32),
            jax.ShapeDtypeStruct((NG, 2), jnp.float32),
        ],
    )(psum, cnt, Wlin, blin)


def kernel(x, edge_index, batch, Wl1, Wr1, att1, b1, gw1, gb1, gm1,
           Wl2, Wr2, att2, b2, gw2, gb2, gm2,
           Wl3, Wr3, att3, b3, gw3, gb3, gm3, Wlin, blin):
    n = x.shape[0]
    e = edge_index.shape[1]
    bn = 2000
    nb = n // bn
    n_pad = -(-n // (NS * 8)) * (NS * 8)
    batch3 = batch.reshape(nb, 1, bn)
    src = edge_index[0]
    dst = edge_index[1]
    eic = jnp.stack([src.reshape(-1, K), dst.reshape(-1, K)], axis=1)
    zs = jnp.zeros((n_pad, L), jnp.float32)

    edge1 = _make_edge_kernel(n_pad, e, store_ex=False)
    edge3 = _make_edge_kernel(n_pad, e, store_ex=True)
    alpha3 = _make_alpha_kernel(n_pad, e)

    tabs = _tables(x, Wl1, Wr1, bn, n_pad)
    layer_params = [
        (att1, b1, gw1, gb1, gm1, Wl2, Wr2),
        (att2, b2, gw2, gb2, gm2, Wl3, Wr3),
        (att3, b3, gw3, gb3, gm3, None, None),
    ]
    ex = dtot = psum = cnt = None
    for li, (att, bb, gw, gb, gm, Wln, Wrn) in enumerate(layer_params):
        last = li == 2
        ek = edge3 if last else edge1
        res = ek(eic, tabs[0], tabs[1], tabs[2], tabs[3],
                 att.reshape(2, L), zs)
        if last:
            s_parts, d_parts, ex0, ex1 = res
        else:
            s_parts, d_parts = res
        d_parts = d_parts.reshape(NC, -1, 2)
        cres = _combine(s_parts, d_parts, bb.reshape(1, HC), batch3, n, bn,
                        relu=not last, want_dtot=last)
        if last:
            t, hsum, cnt, dtot = cres
        else:
            t, hsum, cnt = cres
        y, vsum = _center(t, hsum, cnt, batch3, gm.reshape(1, HC), bn)
        fres = _finish(y, vsum, cnt, batch3, gw.reshape(1, HC),
                       gb.reshape(1, HC), Wln, Wrn, bn, n_pad)
        if last:
            (psum,) = fres
        else:
            tabs = fres

    a3 = alpha3(dst, ex0, ex1, dtot)
    pooled, o = _head(psum, cnt, Wlin, blin.reshape(1, 2))
    return (o, pooled, a3)


# speedup vs baseline: 50.5960x; 1.0011x over previous
import jax
import jax.numpy as jnp
from jax import lax
from jax.experimental import pallas as pl
from jax.experimental.pallas import tpu as pltpu
from jax.experimental.pallas import tpu_sc as plsc

NC, NS, L = 2, 16, 16
NW = NC * NS
NG = 64
HC = 32
K = 80


def _make_edge_kernel(n, e, store_ex):
    epw = e // NW
    nchunks = epw // K
    npt = n // NS
    G = K // L
    n8 = -(-n // (8 * NS * 8)) * (NS * 8)
    npt8 = n8 // NS
    npairs = (nchunks + 1) // 2
    assert n % (NS * 8) == 0 and epw % K == 0
    mesh = plsc.VectorSubcoreMesh(core_axis_name="c", subcore_axis_name="s",
                                  num_cores=NC, num_subcores=NS)

    out_type = [
        jax.ShapeDtypeStruct((NC, 2, n, L), jnp.float32),
        jax.ShapeDtypeStruct((NC, n8, L), jnp.float32),
    ]
    if store_ex:
        out_type.append(jax.ShapeDtypeStruct((e,), jnp.float32))
        out_type.append(jax.ShapeDtypeStruct((e,), jnp.float32))

    scratch = [
        pltpu.VMEM((2, K), jnp.int32),
        pltpu.VMEM((2, K), jnp.int32),
        pltpu.VMEM((K,), jnp.int32),
        pltpu.VMEM((K,), jnp.int32),
        pltpu.VMEM((K,), jnp.int32),
        pltpu.VMEM((K,), jnp.int32),
        pltpu.VMEM((2, K, L), jnp.float32),
        pltpu.VMEM((2, K, L), jnp.float32),
        pltpu.VMEM((2, K, L), jnp.float32),
        pltpu.VMEM((2, K, L), jnp.float32),
        pltpu.VMEM((2, K), jnp.float32),
        pltpu.VMEM((2, L), jnp.float32),
        pltpu.VMEM_SHARED((n, L), jnp.float32),
        pltpu.VMEM_SHARED((n8, L), jnp.float32),
        pltpu.SemaphoreType.DMA,
        pltpu.SemaphoreType.DMA,
    ]

    def body(*refs):
        if store_ex:
            (eic, xl0, xl1, xr0, xr1, att, zs, s_out, d_out,
             ex0_out, ex1_out,
             eiA, eiB, dwA, dwB, dhA, dhB, xl_v, xr_v, s_v, d_v, ex_v, att_v,
             s_sh, d_sh, sg0, sg1) = refs
            ex_outs = (ex0_out, ex1_out)
        else:
            (eic, xl0, xl1, xr0, xr1, att, zs, s_out, d_out,
             eiA, eiB, dwA, dwB, dhA, dhB, xl_v, xr_v, s_v, d_v, ex_v, att_v,
             s_sh, d_sh, sg0, sg1) = refs
            ex_outs = None
        sem_g = (sg0, sg1)
        ei = (eiA, eiB)
        dw = (dwA, dwB)
        dh = (dhA, dhB)
        cid = lax.axis_index("c")
        sid = lax.axis_index("s")
        wid = cid * NS + sid
        row0 = pl.multiple_of(sid * npt, 8)
        row8 = pl.multiple_of(sid * npt8, 8)

        pltpu.sync_copy(att, att_v)
        pltpu.sync_copy(zs.at[pl.ds(row0, npt), :], s_sh.at[pl.ds(row0, npt), :])
        pltpu.sync_copy(zs.at[pl.ds(0, npt8), :], d_sh.at[pl.ds(row8, npt8), :])

        rows0 = lax.iota(jnp.int32, L)
        zero16 = jnp.zeros((L,), jnp.float32)
        plsc.subcore_barrier()

        for h in range(2):
            xlt = xl0 if h == 0 else xl1
            xrt = xr0 if h == 0 else xr1
            att_row = att_v[h, :]
            att_s = [att_row[c] for c in range(L)]

            def fetch_and_gather(c, b, xlt=xlt, xrt=xrt):
                cix = wid * nchunks + c
                pltpu.sync_copy(eic.at[cix], ei[b])
                for g in range(G):
                    dw[b][pl.ds(g * L, L)] = ei[b][1, pl.ds(g * L, L)]
                pltpu.async_copy(xlt.at[ei[b].at[0]], xl_v.at[b], sem_g[b])
                pltpu.async_copy(xrt.at[dw[b]], xr_v.at[b], sem_g[b])

            def drain_g(b, xlt=xlt, xrt=xrt):
                pltpu.make_async_copy(xlt.at[ei[b].at[0]], xl_v.at[b],
                                      sem_g[b]).wait()
                pltpu.make_async_copy(xrt.at[dw[b]], xr_v.at[b],
                                      sem_g[b]).wait()

            def issue_scatters(c, b, h=h):
                pltpu.sync_copy(s_v.at[b], s_sh.at[dw[b]], add=True)
                pltpu.sync_copy(d_v.at[b], d_sh.at[dh[b]], add=True)
                if store_ex:
                    base = pl.multiple_of(wid * epw + c * K, 8)
                    pltpu.sync_copy(ex_v.at[b], ex_outs[h].at[pl.ds(base, K)])

            def compute(c, b, h=h, att_s=att_s):
                for jj in range(K):
                    d_v[b, jj, :] = zero16
                for g in range(G):
                    ridx = rows0 + (g * L)
                    acc = zero16
                    cols = []
                    for c16 in range(L):
                        cc = jnp.full((L,), c16, jnp.int32)
                        a = plsc.load_gather(xl_v.at[b], [ridx, cc])
                        bb = plsc.load_gather(xr_v.at[b], [ridx, cc])
                        z = a + bb
                        zl = jnp.maximum(z, 0.2 * z)
                        acc = acc + zl * att_s[c16]
                        cols.append(a)
                    ex = jnp.exp(acc)
                    ex_v[b, pl.ds(g * L, L)] = ex
                    for c16 in range(L):
                        cc = jnp.full((L,), c16, jnp.int32)
                        plsc.store_scatter(s_v.at[b], [ridx, cc], ex * cols[c16])
                    dvec = dw[b][pl.ds(g * L, L)]
                    colv = ((dvec & 7) << 1) + h
                    plsc.store_scatter(d_v.at[b], [ridx, colv], ex)
                    dh[b][pl.ds(g * L, L)] = dvec >> 3

            def half(c, b):
                drain_g(b)
                compute(c, b)

                @pl.when(c + 1 < nchunks)
                def _():
                    fetch_and_gather(c + 1, 1 - b)

                issue_scatters(c, b)

            def pair(j, _):
                c0 = 2 * j
                half(c0, 0)

                @pl.when(c0 + 1 < nchunks)
                def _():
                    half(c0 + 1, 1)

                return 0

            fetch_and_gather(0, 0)
            lax.fori_loop(0, npairs, pair, 0)
            plsc.subcore_barrier()
            pltpu.sync_copy(s_sh.at[pl.ds(row0, npt), :],
                            s_out.at[cid, h, pl.ds(row0, npt), :])
            plsc.subcore_barrier()
            if h == 0:
                pltpu.sync_copy(zs.at[pl.ds(row0, npt), :],
                                s_sh.at[pl.ds(row0, npt), :])
                plsc.subcore_barrier()
        pltpu.sync_copy(d_sh.at[pl.ds(row8, npt8), :],
                        d_out.at[cid, pl.ds(row8, npt8), :])

    return pl.kernel(body, out_type=out_type, mesh=mesh,
                     scratch_types=scratch,
                     compiler_params=pltpu.CompilerParams(
                         needs_layout_passes=False,
                         use_tc_tiling_on_sc=False))


def _make_alpha_kernel(n, e):
    epw = e // NW
    nchunks = epw // K
    G = K // L
    mesh = plsc.VectorSubcoreMesh(core_axis_name="c", subcore_axis_name="s",
                                  num_cores=NC, num_subcores=NS)

    scratch = [
        pltpu.VMEM((K,), jnp.int32),
        pltpu.VMEM((K, L), jnp.float32),
        pltpu.VMEM((K,), jnp.float32),
        pltpu.VMEM((K,), jnp.float32),
        pltpu.VMEM((K, 2), jnp.float32),
        pltpu.SemaphoreType.DMA,
    ]

    def body(dst_h, ex0_hbm, ex1_hbm, den_hbm, a3_out,
             dst_v, den_v, e0_v, e1_v, al_v, sem):
        cid = lax.axis_index("c")
        sid = lax.axis_index("s")
        wid = cid * NS + sid
        rows0 = lax.iota(jnp.int32, L)

        def chunk(i, _):
            base = pl.multiple_of(wid * epw + i * K, 8)
            pltpu.sync_copy(dst_h.at[pl.ds(base, K)], dst_v)
            cp = pltpu.async_copy(den_hbm.at[dst_v], den_v, sem)
            pltpu.sync_copy(ex0_hbm.at[pl.ds(base, K)], e0_v)
            pltpu.sync_copy(ex1_hbm.at[pl.ds(base, K)], e1_v)
            cp.wait()
            for g in range(G):
                ridx = rows0 + (g * L)
                for h in range(2):
                    hh = jnp.full((L,), h, jnp.int32)
                    den = plsc.load_gather(den_v, [ridx, hh])
                    ev = e0_v if h == 0 else e1_v
                    exg = ev[pl.ds(g * L, L)]
                    al = exg / (den + 1e-16)
                    plsc.store_scatter(al_v, [ridx, hh], al)
            pltpu.sync_copy(al_v, a3_out.at[pl.ds(base, K), :])
            return 0

        lax.fori_loop(0, nchunks, chunk, 0)

    return pl.kernel(
        body, out_type=jax.ShapeDtypeStruct((e, 2), jnp.float32),
        mesh=mesh, scratch_types=scratch,
        compiler_params=pltpu.CompilerParams(
            needs_layout_passes=False, use_tc_tiling_on_sc=False))


def _tables(xin, Wl, Wr, bn, n_pad):
    n, din = xin.shape
    nb = n // bn

    def body(x_ref, wl_ref, wr_ref, o1, o2, o3, o4):
        xl = jnp.dot(x_ref[...], wl_ref[...], preferred_element_type=jnp.float32,
                 precision=lax.Precision.HIGHEST)
        xr = jnp.dot(x_ref[...], wr_ref[...], preferred_element_type=jnp.float32,
                 precision=lax.Precision.HIGHEST)
        o1[...] = xl[:, :L]
        o2[...] = xl[:, L:]
        o3[...] = xr[:, :L]
        o4[...] = xr[:, L:]

    return pl.pallas_call(
        body,
        grid=(nb,),
        in_specs=[
            pl.BlockSpec((bn, din), lambda j: (j, 0)),
            pl.BlockSpec((din, HC), lambda j: (0, 0)),
            pl.BlockSpec((din, HC), lambda j: (0, 0)),
        ],
        out_specs=[pl.BlockSpec((bn, L), lambda j: (j, 0))] * 4,
        out_shape=[jax.ShapeDtypeStruct((n_pad, L), jnp.float32)] * 4,
    )(xin, Wl, Wr)


def _onehot(bt):
    return (bt[:, None] == lax.broadcasted_iota(jnp.int32, (1, NG), 1)
            ).astype(jnp.float32)


def _combine(s_parts, d_parts, bias2, batch3, n, bn, relu, want_dtot):
    n_pad = s_parts.shape[2]
    nb = n // bn
    outs = [
        jax.ShapeDtypeStruct((n, HC), jnp.float32),
        jax.ShapeDtypeStruct((NG, HC), jnp.float32),
        jax.ShapeDtypeStruct((NG, HC), jnp.float32),
    ]
    if want_dtot:
        outs.append(jax.ShapeDtypeStruct((n_pad, L), jnp.float32))

    def body(s_ref, d_ref, b_ref, bt_ref, t_out, hsum_out, cnt_out,
             *maybe_dtot):
        j = pl.program_id(0)
        d = d_ref[0] + d_ref[1]
        parts = []
        for h in range(2):
            sh = s_ref[0, h] + s_ref[1, h]
            parts.append(sh / (d[:, h:h + 1] + 1e-16))
        t = jnp.concatenate(parts, axis=1) + b_ref[...]
        if relu:
            t = jnp.maximum(t, 0.0)
        t_out[...] = t
        oh = _onehot(bt_ref[0, 0])

        @pl.when(j == 0)
        def _():
            hsum_out[...] = jnp.zeros_like(hsum_out)
            cnt_out[...] = jnp.zeros_like(cnt_out)

        hsum_out[...] += jnp.dot(oh.T, t, preferred_element_type=jnp.float32,
                 precision=lax.Precision.HIGHEST)
        cnt_out[...] += jnp.dot(
            oh.T, jnp.ones((bn, HC), jnp.float32),
            preferred_element_type=jnp.float32,
                 precision=lax.Precision.HIGHEST)
        if want_dtot:
            maybe_dtot[0][...] = jnp.concatenate(
                [d, jnp.zeros((d.shape[0], L - 2), jnp.float32)], axis=1)

    out_specs = [
        pl.BlockSpec((bn, HC), lambda j: (j, 0)),
        pl.BlockSpec((NG, HC), lambda j: (0, 0)),
        pl.BlockSpec((NG, HC), lambda j: (0, 0)),
    ]
    if want_dtot:
        out_specs.append(pl.BlockSpec((bn, L), lambda j: (j, 0)))

    return pl.pallas_call(
        body,
        grid=(nb,),
        in_specs=[
            pl.BlockSpec((NC, 2, bn, L), lambda j: (0, 0, j, 0)),
            pl.BlockSpec((NC, bn, 2), lambda j: (0, j, 0)),
            pl.BlockSpec((1, HC), lambda j: (0, 0)),
            pl.BlockSpec((1, 1, bn), lambda j: (j, 0, 0)),
        ],
        out_specs=out_specs,
        out_shape=outs,
    )(s_parts, d_parts, bias2, batch3)


def _center(t, hsum, cnt, batch3, gm2, bn):
    n = t.shape[0]
    nb = n // bn

    def body(t_ref, hsum_ref, cnt_ref, bt_ref, gm_ref, y_out, vsum_out):
        j = pl.program_id(0)
        cnt = jnp.maximum(cnt_ref[...], 1.0)
        mean = hsum_ref[...] / cnt
        oh = _onehot(bt_ref[0, 0])
        mean_n = jnp.dot(oh, mean, preferred_element_type=jnp.float32,
                 precision=lax.Precision.HIGHEST)
        y = t_ref[...] - mean_n * gm_ref[...]
        y_out[...] = y

        @pl.when(j == 0)
        def _():
            vsum_out[...] = jnp.zeros_like(vsum_out)

        vsum_out[...] += jnp.dot(oh.T, y * y,
                                 preferred_element_type=jnp.float32,
                 precision=lax.Precision.HIGHEST)

    return pl.pallas_call(
        body,
        grid=(nb,),
        in_specs=[
            pl.BlockSpec((bn, HC), lambda j: (j, 0)),
            pl.BlockSpec((NG, HC), lambda j: (0, 0)),
            pl.BlockSpec((NG, HC), lambda j: (0, 0)),
            pl.BlockSpec((1, 1, bn), lambda j: (j, 0, 0)),
            pl.BlockSpec((1, HC), lambda j: (0, 0)),
        ],
        out_specs=[
            pl.BlockSpec((bn, HC), lambda j: (j, 0)),
            pl.BlockSpec((NG, HC), lambda j: (0, 0)),
        ],
        out_shape=[
            jax.ShapeDtypeStruct((n, HC), jnp.float32),
            jax.ShapeDtypeStruct((NG, HC), jnp.float32),
        ],
    )(t, hsum, cnt, batch3, gm2)


def _finish(y, vsum, cnt, batch3, gw2, gb2, Wl_next, Wr_next, bn, n_pad):
    n = y.shape[0]
    nb = n // bn
    last = Wl_next is None

    def body(y_ref, vsum_ref, cnt_ref, bt_ref, gw_ref, gb_ref, *rest):
        j = pl.program_id(0)
        cnt = jnp.maximum(cnt_ref[...], 1.0)
        var = vsum_ref[...] / cnt
        r = lax.rsqrt(var + 1e-5)
        oh = _onehot(bt_ref[0, 0])
        rn = jnp.dot(oh, r, preferred_element_type=jnp.float32,
                 precision=lax.Precision.HIGHEST)
        res = y_ref[...] * rn * gw_ref[...] + gb_ref[...]
        if last:
            (psum_out,) = rest

            @pl.when(j == 0)
            def _():
                psum_out[...] = jnp.zeros_like(psum_out)

            psum_out[...] += jnp.dot(oh.T, res,
                                     preferred_element_type=jnp.float32,
                 precision=lax.Precision.HIGHEST)
        else:
            wl_ref, wr_ref, o1, o2, o3, o4 = rest
            xl = jnp.dot(res, wl_ref[...], preferred_element_type=jnp.float32,
                 precision=lax.Precision.HIGHEST)
            xr = jnp.dot(res, wr_ref[...], preferred_element_type=jnp.float32,
                 precision=lax.Precision.HIGHEST)
            o1[...] = xl[:, :L]
            o2[...] = xl[:, L:]
            o3[...] = xr[:, :L]
            o4[...] = xr[:, L:]

    in_specs = [
        pl.BlockSpec((bn, HC), lambda j: (j, 0)),
        pl.BlockSpec((NG, HC), lambda j: (0, 0)),
        pl.BlockSpec((NG, HC), lambda j: (0, 0)),
        pl.BlockSpec((1, 1, bn), lambda j: (j, 0, 0)),
        pl.BlockSpec((1, HC), lambda j: (0, 0)),
        pl.BlockSpec((1, HC), lambda j: (0, 0)),
    ]
    args = [y, vsum, cnt, batch3, gw2, gb2]
    if last:
        out_specs = [pl.BlockSpec((NG, HC), lambda j: (0, 0))]
        out_shape = [jax.ShapeDtypeStruct((NG, HC), jnp.float32)]
    else:
        in_specs += [pl.BlockSpec((HC, HC), lambda j: (0, 0))] * 2
        args += [Wl_next, Wr_next]
        out_specs = [pl.BlockSpec((bn, L), lambda j: (j, 0))] * 4
        out_shape = [jax.ShapeDtypeStruct((n_pad, L), jnp.float32)] * 4

    return pl.pallas_call(body, grid=(nb,), in_specs=in_specs,
                          out_specs=out_specs, out_shape=out_shape)(*args)


def _head(psum, cnt, Wlin, blin):
    def body(p_ref, c_ref, w_ref, b_ref, pooled_out, o_out):
        pooled = p_ref[...] / jnp.maximum(c_ref[...], 1.0)
        pooled_out[...] = pooled
        o_out[...] = jnp.dot(pooled, w_ref[...],
                             preferred_element_type=jnp.float32,
                 precision=lax.Precision.HIGHEST) + b_ref[...]

    return pl.pallas_call(
        body,
        in_specs=[
            pl.BlockSpec((NG, HC), lambda: (0, 0)),
            pl.BlockSpec((NG, HC), lambda: (0, 0)),
            pl.BlockSpec((HC, 2), lambda: (0, 0)),
            pl.BlockSpec((1, 2), lambda: (0, 0)),
        ],
        out_specs=[
            pl.BlockSpec((NG, HC), lambda: (0, 0)),
            pl.BlockSpec((NG, 2), lambda: (0, 0)),
        ],
        out_shape=[
            jax.ShapeDtypeStruct((NG, HC), jnp.float32),
            jax.ShapeDtypeStruct((NG, 2), jnp.float32),
        ],
    )(psum, cnt, Wlin, blin)


def kernel(x, edge_index, batch, Wl1, Wr1, att1, b1, gw1, gb1, gm1,
           Wl2, Wr2, att2, b2, gw2, gb2, gm2,
           Wl3, Wr3, att3, b3, gw3, gb3, gm3, Wlin, blin):
    n = x.shape[0]
    e = edge_index.shape[1]
    bn = 2000
    nb = n // bn
    n_pad = -(-n // (NS * 8)) * (NS * 8)
    batch3 = batch.reshape(nb, 1, bn)
    src = edge_index[0]
    dst = edge_index[1]
    eic = jnp.stack([src.reshape(-1, K), dst.reshape(-1, K)], axis=1)
    zs = jnp.zeros((n_pad, L), jnp.float32)

    edge1 = _make_edge_kernel(n_pad, e, store_ex=False)
    edge3 = _make_edge_kernel(n_pad, e, store_ex=True)
    alpha3 = _make_alpha_kernel(n_pad, e)

    tabs = _tables(x, Wl1, Wr1, bn, n_pad)
    layer_params = [
        (att1, b1, gw1, gb1, gm1, Wl2, Wr2),
        (att2, b2, gw2, gb2, gm2, Wl3, Wr3),
        (att3, b3, gw3, gb3, gm3, None, None),
    ]
    ex = dtot = psum = cnt = None
    for li, (att, bb, gw, gb, gm, Wln, Wrn) in enumerate(layer_params):
        last = li == 2
        ek = edge3 if last else edge1
        res = ek(eic, tabs[0], tabs[1], tabs[2], tabs[3],
                 att.reshape(2, L), zs)
        if last:
            s_parts, d_parts, ex0, ex1 = res
        else:
            s_parts, d_parts = res
        d_parts = d_parts.reshape(NC, -1, 2)
        cres = _combine(s_parts, d_parts, bb.reshape(1, HC), batch3, n, bn,
                        relu=not last, want_dtot=last)
        if last:
            t, hsum, cnt, dtot = cres
        else:
            t, hsum, cnt = cres
        y, vsum = _center(t, hsum, cnt, batch3, gm.reshape(1, HC), bn)
        fres = _finish(y, vsum, cnt, batch3, gw.reshape(1, HC),
                       gb.reshape(1, HC), Wln, Wrn, bn, n_pad)
        if last:
            (psum,) = fres
        else:
            tabs = fres

    a3 = alpha3(dst, ex0, ex1, dtot)
    pooled, o = _head(psum, cnt, Wlin, blin.reshape(1, 2))
    return (o, pooled, a3)
